# baseline (speedup 1.0000x reference)
import functools

import jax
import jax.numpy as jnp
from jax import lax
from jax.experimental import pallas as pl
from jax.experimental.pallas import tpu as pltpu
from jax.experimental.pallas import tpu_sc as plsc

VOCAB = 1000000
MAXLEN = 200
EMBED_DIM = 64
BATCH = 4096

NUM_CORES = 2
NUM_SUBCORES = 16
NUM_WORKERS = NUM_CORES * NUM_SUBCORES
BBLK = 128
NBT = BATCH // BBLK
CQ = EMBED_DIM // 8
L = 16
NG = BBLK // L
PROW = EMBED_DIM * L


@functools.partial(
    pl.kernel,
    out_type=jax.ShapeDtypeStruct((MAXLEN, CQ, NBT, 8 * BBLK), jnp.float32),
    mesh=plsc.VectorSubcoreMesh(core_axis_name="c", subcore_axis_name="s"),
    scratch_types=[pltpu.VMEM((MAXLEN, BBLK), jnp.int32)]
    + [pltpu.VMEM((PROW,), jnp.float32) for _ in range(2)]
    + [pltpu.VMEM((BBLK,), jnp.int32) for _ in range(2)]
    + [pltpu.VMEM((BBLK, BBLK), jnp.float32) for _ in range(2)]
    + [pltpu.VMEM((CQ, 8 * BBLK), jnp.float32) for _ in range(2)]
    + [pltpu.SemaphoreType.DMA for _ in range(7)],
    compiler_params=pltpu.CompilerParams(needs_layout_passes=False),
)
def _emb_kernel(x2, tok, posB, out_hbm,
                xcol, pb0, pb1, qb0, qb1, r0, r1, o0, o1,
                xs, gs0, gs1, ps0, ps1, os0, os1):
    pb = (pb0, pb1)
    qb = (qb0, qb1)
    rows = (r0, r1)
    outs = (o0, o1)
    gsem = (gs0, gs1)
    psem = (ps0, ps1)
    osem = (os0, os1)

    wid = lax.axis_index("s") * NUM_CORES + lax.axis_index("c")
    b0 = wid * BBLK

    pltpu.async_copy(x2.at[:, pl.ds(b0, BBLK)], xcol, xs)
    pltpu.make_async_copy(x2.at[:, pl.ds(0, BBLK)], xcol, xs).wait()

    def fire(t, p):
        pltpu.async_copy(posB.at[pl.ds(t * PROW, PROW)], pb[p], psem[p])
        for g in range(NG):
            sl = pl.ds(g * L, L)
            qb[p][sl] = lax.shift_right_logical(xcol[t, sl], 1)
        pltpu.async_copy(tok.at[qb[p]], rows[p], gsem[p])

    def wait_in(p):
        pltpu.make_async_copy(tok.at[pl.ds(0, BBLK)], rows[p], gsem[p]).wait()
        pltpu.make_async_copy(posB.at[pl.ds(0, PROW)], pb[p], psem[p]).wait()

    def fire_out(t, p):
        pltpu.async_copy(outs[p], out_hbm.at[t, :, wid], osem[p])

    def wait_out(p):
        pltpu.make_async_copy(outs[p], out_hbm.at[0, :, 0], osem[p]).wait()

    riv = [lax.iota(jnp.int32, L) + g * L for g in range(NG)]
    ones = jnp.full((L,), 1, jnp.int32)

    def transpose_add(t, p):
        r = rows[p]
        ob = outs[p]
        pbuf = pb[p]
        hv = [
            lax.shift_left(
                lax.bitwise_and(xcol[t, pl.ds(g * L, L)], 1), 6
            )
            for g in range(NG)
        ]

        def cbody(c, civ):
            cq = lax.shift_right_logical(c, 3)
            off = lax.mul(lax.bitwise_and(c, 7), BBLK)
            pv = pbuf[pl.ds(c * L, L)]
            for g in range(NG):
                vals = plsc.load_gather(r, [riv[g], hv[g] + civ])
                ob[cq, pl.ds(off + g * L, L)] = vals + pv
            return civ + ones

        lax.fori_loop(0, EMBED_DIM, cbody, jnp.full((L,), 0, jnp.int32))

    fire(0, 0)
    for t in (0, 1):
        p = t % 2
        fire(t + 1, 1 - p)
        wait_in(p)
        transpose_add(t, p)
        fire_out(t, p)

    def macro(m, carry):
        for par in range(2):
            t = 2 * m + par
            fire(t + 1, 1 - par)
            wait_in(par)
            wait_out(par)
            transpose_add(t, par)
            fire_out(t, par)
        return carry

    lax.fori_loop(1, MAXLEN // 2 - 1, macro, 0)

    for t in (MAXLEN - 2, MAXLEN - 1):
        p = t % 2
        if t + 1 < MAXLEN:
            fire(t + 1, 1 - p)
        wait_in(p)
        wait_out(p)
        transpose_add(t, p)
        fire_out(t, p)
    wait_out(0)
    wait_out(1)


def kernel(x, token_emb, pos_emb):
    x2 = jnp.transpose(x).astype(jnp.int32)
    posB = jnp.broadcast_to(
        pos_emb[:, :, None], (MAXLEN, EMBED_DIM, L)
    ).reshape(-1)
    tok2 = token_emb.reshape(VOCAB // 2, 2 * EMBED_DIM)
    y = _emb_kernel(x2, tok2, posB)
    y5 = y.reshape(MAXLEN, CQ, NBT, 8, BBLK)
    z = jnp.transpose(y5, (2, 4, 0, 1, 3))
    return z.reshape(BATCH, MAXLEN, EMBED_DIM)


# speedup vs baseline: 1.4634x; 1.4634x over previous
import functools

import jax
import jax.numpy as jnp
from jax import lax
from jax.experimental import pallas as pl
from jax.experimental.pallas import tpu as pltpu
from jax.experimental.pallas import tpu_sc as plsc

VOCAB = 1000000
MAXLEN = 200
EMBED_DIM = 64
BATCH = 4096

NUM_CORES = 2
NUM_SUBCORES = 16
NUM_WORKERS = NUM_CORES * NUM_SUBCORES
BBLK = 128
NBT = BATCH // BBLK
CQ = EMBED_DIM // 8
L = 16
NG = BBLK // L
PROW = EMBED_DIM * L


@functools.partial(
    pl.kernel,
    out_type=jax.ShapeDtypeStruct((MAXLEN, CQ, NBT, 8 * BBLK), jnp.float32),
    mesh=plsc.VectorSubcoreMesh(core_axis_name="c", subcore_axis_name="s"),
    scratch_types=[pltpu.VMEM((MAXLEN, BBLK), jnp.int32)]
    + [pltpu.VMEM((PROW,), jnp.float32) for _ in range(2)]
    + [pltpu.VMEM((BBLK,), jnp.int32) for _ in range(2)]
    + [pltpu.VMEM((BBLK, BBLK), jnp.float32) for _ in range(2)]
    + [pltpu.VMEM((CQ, 8 * BBLK), jnp.float32) for _ in range(2)]
    + [pltpu.SemaphoreType.DMA for _ in range(7)],
    compiler_params=pltpu.CompilerParams(needs_layout_passes=False),
)
def _emb_kernel(x2, tok, posB, out_hbm,
                xcol, pb0, pb1, qb0, qb1, r0, r1, o0, o1,
                xs, gs0, gs1, ps0, ps1, os0, os1):
    pb = (pb0, pb1)
    qb = (qb0, qb1)
    rows = (r0, r1)
    outs = (o0, o1)
    gsem = (gs0, gs1)
    psem = (ps0, ps1)
    osem = (os0, os1)

    wid = lax.axis_index("s") * NUM_CORES + lax.axis_index("c")
    b0 = wid * BBLK

    pltpu.async_copy(x2.at[:, pl.ds(b0, BBLK)], xcol, xs)
    pltpu.make_async_copy(x2.at[:, pl.ds(0, BBLK)], xcol, xs).wait()

    def fire(t, p):
        pltpu.async_copy(posB.at[pl.ds(t * PROW, PROW)], pb[p], psem[p])
        for g in range(NG):
            sl = pl.ds(g * L, L)
            qb[p][sl] = lax.shift_right_logical(xcol[t, sl], 1)
        pltpu.async_copy(tok.at[qb[p]], rows[p], gsem[p])

    def wait_in(p):
        pltpu.make_async_copy(tok.at[pl.ds(0, BBLK)], rows[p], gsem[p]).wait()
        pltpu.make_async_copy(posB.at[pl.ds(0, PROW)], pb[p], psem[p]).wait()

    def fire_out(t, p):
        pltpu.async_copy(outs[p], out_hbm.at[t, :, wid], osem[p])

    def wait_out(p):
        pltpu.make_async_copy(outs[p], out_hbm.at[0, :, 0], osem[p]).wait()

    riv = [lax.iota(jnp.int32, L) + g * L for g in range(NG)]
    ones = jnp.full((L,), 1, jnp.int32)

    def transpose_add(t, p):
        r = rows[p]
        ob = outs[p]
        pbuf = pb[p]
        hv = [
            lax.shift_left(
                lax.bitwise_and(xcol[t, pl.ds(g * L, L)], 1), 6
            )
            for g in range(NG)
        ]

        @plsc.parallel_loop(0, EMBED_DIM, unroll=8,
                            carry=jnp.full((L,), 0, jnp.int32))
        def cbody(c, civ):
            cq = lax.shift_right_logical(c, 3)
            off = lax.mul(lax.bitwise_and(c, 7), BBLK)
            pv = pbuf[pl.ds(c * L, L)]
            for g in range(NG):
                vals = plsc.load_gather(r, [riv[g], hv[g] + civ])
                ob[cq, pl.ds(off + g * L, L)] = vals + pv
            return civ + ones

    fire(0, 0)
    for t in (0, 1):
        p = t % 2
        fire(t + 1, 1 - p)
        wait_in(p)
        transpose_add(t, p)
        fire_out(t, p)

    def macro(m, carry):
        for par in range(2):
            t = 2 * m + par
            fire(t + 1, 1 - par)
            wait_in(par)
            wait_out(par)
            transpose_add(t, par)
            fire_out(t, par)
        return carry

    lax.fori_loop(1, MAXLEN // 2 - 1, macro, 0)

    for t in (MAXLEN - 2, MAXLEN - 1):
        p = t % 2
        if t + 1 < MAXLEN:
            fire(t + 1, 1 - p)
        wait_in(p)
        wait_out(p)
        transpose_add(t, p)
        fire_out(t, p)
    wait_out(0)
    wait_out(1)


def kernel(x, token_emb, pos_emb):
    x2 = jnp.transpose(x).astype(jnp.int32)
    posB = jnp.broadcast_to(
        pos_emb[:, :, None], (MAXLEN, EMBED_DIM, L)
    ).reshape(-1)
    tok2 = token_emb.reshape(VOCAB // 2, 2 * EMBED_DIM)
    y = _emb_kernel(x2, tok2, posB)
    y5 = y.reshape(MAXLEN, CQ, NBT, 8, BBLK)
    z = jnp.transpose(y5, (2, 4, 0, 1, 3))
    return z.reshape(BATCH, MAXLEN, EMBED_DIM)


# speedup vs baseline: 1.6285x; 1.1128x over previous
import functools

import jax
import jax.numpy as jnp
from jax import lax
from jax.experimental import pallas as pl
from jax.experimental.pallas import tpu as pltpu
from jax.experimental.pallas import tpu_sc as plsc

VOCAB = 1000000
MAXLEN = 200
EMBED_DIM = 64
BATCH = 4096

NUM_CORES = 2
NUM_SUBCORES = 16
NUM_WORKERS = NUM_CORES * NUM_SUBCORES
BBLK = 128
NBT = BATCH // BBLK
CQ = EMBED_DIM // 8
L = 16
NG = BBLK // L
PROW = EMBED_DIM * L


@functools.partial(
    pl.kernel,
    out_type=jax.ShapeDtypeStruct((MAXLEN, CQ, NBT, 8, BBLK), jnp.float32),
    mesh=plsc.VectorSubcoreMesh(core_axis_name="c", subcore_axis_name="s"),
    scratch_types=[pltpu.VMEM((MAXLEN, BBLK), jnp.int32)]
    + [pltpu.VMEM((PROW,), jnp.float32) for _ in range(2)]
    + [pltpu.VMEM((BBLK,), jnp.int32) for _ in range(2)]
    + [pltpu.VMEM((BBLK, BBLK), jnp.float32) for _ in range(2)]
    + [pltpu.VMEM((CQ, 8, BBLK), jnp.float32) for _ in range(2)]
    + [pltpu.SemaphoreType.DMA for _ in range(7)],
    compiler_params=pltpu.CompilerParams(needs_layout_passes=False),
)
def _emb_kernel(x2, tok, posB, out_hbm,
                xcol, pb0, pb1, qb0, qb1, r0, r1, o0, o1,
                xs, gs0, gs1, ps0, ps1, os0, os1):
    pb = (pb0, pb1)
    qb = (qb0, qb1)
    rows = (r0, r1)
    outs = (o0, o1)
    gsem = (gs0, gs1)
    psem = (ps0, ps1)
    osem = (os0, os1)

    wid = lax.axis_index("s") * NUM_CORES + lax.axis_index("c")
    b0 = wid * BBLK

    pltpu.async_copy(x2.at[:, pl.ds(b0, BBLK)], xcol, xs)
    pltpu.make_async_copy(x2.at[:, pl.ds(0, BBLK)], xcol, xs).wait()

    def fire(t, p):
        pltpu.async_copy(posB.at[pl.ds(t * PROW, PROW)], pb[p], psem[p])
        for g in range(NG):
            sl = pl.ds(g * L, L)
            qb[p][sl] = lax.shift_right_logical(xcol[t, sl], 1)
        pltpu.async_copy(tok.at[qb[p]], rows[p], gsem[p])

    def wait_in(p):
        pltpu.make_async_copy(tok.at[pl.ds(0, BBLK)], rows[p], gsem[p]).wait()
        pltpu.make_async_copy(posB.at[pl.ds(0, PROW)], pb[p], psem[p]).wait()

    def fire_out(t, p):
        pltpu.async_copy(outs[p], out_hbm.at[t, :, wid], osem[p])

    def wait_out(p):
        pltpu.make_async_copy(outs[p], out_hbm.at[0, :, 0], osem[p]).wait()

    riv = [lax.iota(jnp.int32, L) + g * L for g in range(NG)]
    ones = jnp.full((L,), 1, jnp.int32)

    def transpose_add(t, p):
        r = rows[p]
        ob = outs[p]
        pbuf = pb[p]
        hv = [
            lax.shift_left(
                lax.bitwise_and(xcol[t, pl.ds(g * L, L)], 1), 6
            )
            for g in range(NG)
        ]

        @plsc.parallel_loop(0, EMBED_DIM, unroll=8,
                            carry=jnp.full((L,), 0, jnp.int32))
        def cbody(c, civ):
            cq = lax.shift_right_logical(c, 3)
            cr = lax.bitwise_and(c, 7)
            pv = pbuf[pl.ds(c * L, L)]
            for g in range(NG):
                vals = plsc.load_gather(r, [riv[g], hv[g] + civ])
                ob[cq, cr, pl.ds(g * L, L)] = vals + pv
            return civ + ones

    fire(0, 0)
    for t in (0, 1):
        p = t % 2
        fire(t + 1, 1 - p)
        wait_in(p)
        transpose_add(t, p)
        fire_out(t, p)

    def macro(m, carry):
        for par in range(2):
            t = 2 * m + par
            fire(t + 1, 1 - par)
            wait_in(par)
            wait_out(par)
            transpose_add(t, par)
            fire_out(t, par)
        return carry

    lax.fori_loop(1, MAXLEN // 2 - 1, macro, 0)

    for t in (MAXLEN - 2, MAXLEN - 1):
        p = t % 2
        if t + 1 < MAXLEN:
            fire(t + 1, 1 - p)
        wait_in(p)
        wait_out(p)
        transpose_add(t, p)
        fire_out(t, p)
    wait_out(0)
    wait_out(1)


def kernel(x, token_emb, pos_emb):
    x2 = jnp.transpose(x).astype(jnp.int32)
    posB = jnp.broadcast_to(
        pos_emb[:, :, None], (MAXLEN, EMBED_DIM, L)
    ).reshape(-1)
    tok2 = token_emb.reshape(VOCAB // 2, 2 * EMBED_DIM)
    y5 = _emb_kernel(x2, tok2, posB)
    z = jnp.transpose(y5, (2, 4, 0, 1, 3))
    return z.reshape(BATCH, MAXLEN, EMBED_DIM)
